# baseline (speedup 1.0000x reference)
import functools

import jax
import jax.numpy as jnp
from jax import lax
from jax.experimental import pallas as pl
from jax.experimental.pallas import tpu as pltpu
from jax.experimental.pallas import tpu_sc as plsc

_MARGIN = 0.2
_NC, _NS, _L = 2, 16, 16
_NW = _NC * _NS
_C = 320


@functools.lru_cache(maxsize=None)
def _make_sc_kernel(T, D, n_chunks):
    assert n_chunks >= 3
    n_per_w = n_chunks * _C
    mesh = plsc.VectorSubcoreMesh(core_axis_name="c", subcore_axis_name="s")

    @functools.partial(
        pl.kernel,
        out_type=jax.ShapeDtypeStruct((_NW, _L), jnp.float32),
        mesh=mesh,
        compiler_params=pltpu.CompilerParams(needs_layout_passes=False,
                                             use_tc_tiling_on_sc=False),
        scratch_types=[
            pltpu.VMEM((2, _C), jnp.int32),
            pltpu.VMEM((2, _C), jnp.int32),
            pltpu.VMEM((2, _C), jnp.int32),
            pltpu.VMEM((2, _C, D), jnp.float32),
            pltpu.VMEM((2, _C, D), jnp.float32),
            pltpu.VMEM((2, _C, D), jnp.float32),
            pltpu.VMEM((_L,), jnp.float32),
            pltpu.SemaphoreType.DMA,
            pltpu.SemaphoreType.DMA,
        ],
    )
    def tri_loss(emb_hbm, emc_hbm, mom_hbm, ia_hbm, ip_hbm, in_hbm, out_hbm,
                 ia_v, ip_v, in_v, ra_v, rp_v, rn_v, acc_v, sem0, sem1):
        wid = lax.axis_index("s") * _NC + lax.axis_index("c")
        base_w = wid * n_per_w
        lanes = lax.iota(jnp.int32, _L)
        sems = (sem0, sem1)

        def issue(k, b):
            base = base_w + k * _C
            pltpu.sync_copy(ia_hbm.at[pl.ds(base, _C)], ia_v.at[b])
            pltpu.sync_copy(ip_hbm.at[pl.ds(base, _C)], ip_v.at[b])
            pltpu.sync_copy(in_hbm.at[pl.ds(base, _C)], in_v.at[b])
            pltpu.make_async_copy(emb_hbm.at[ia_v.at[b]], ra_v.at[b],
                                  sems[b]).start()
            pltpu.make_async_copy(emc_hbm.at[ip_v.at[b]], rp_v.at[b],
                                  sems[b]).start()
            pltpu.make_async_copy(mom_hbm.at[in_v.at[b]], rn_v.at[b],
                                  sems[b]).start()

        def wait(b):
            pltpu.make_async_copy(emb_hbm.at[ia_v.at[b]], ra_v.at[b],
                                  sems[b]).wait()
            pltpu.make_async_copy(emc_hbm.at[ip_v.at[b]], rp_v.at[b],
                                  sems[b]).wait()
            pltpu.make_async_copy(mom_hbm.at[in_v.at[b]], rn_v.at[b],
                                  sems[b]).wait()

        def compute(k, b, acc):
            base = base_w + k * _C
            ra, rp, rn = ra_v.at[b], rp_v.at[b], rn_v.at[b]

            def group_body(g, acc):
                row = g * _L + lanes
                ap = [jnp.zeros((_L,), jnp.float32) for _ in range(4)]
                an = [jnp.zeros((_L,), jnp.float32) for _ in range(4)]
                for d in range(D):
                    didx = (lanes + d) & (D - 1)
                    va = plsc.load_gather(ra, [row, didx])
                    vp = plsc.load_gather(rp, [row, didx])
                    vn = plsc.load_gather(rn, [row, didx])
                    dp = va - vp
                    dn = va - vn
                    j = d & 3
                    ap[j] = ap[j] + dp * dp
                    an[j] = an[j] + dn * dn
                dd = ((ap[0] - an[0]) + (ap[1] - an[1])) + \
                     ((ap[2] - an[2]) + (ap[3] - an[3]))
                dloss = jnp.maximum(dd + _MARGIN, 0.0)
                valid = (base + row) < T
                return acc + jnp.where(valid, dloss, 0.0)

            return lax.fori_loop(0, _C // _L, group_body, acc)

        issue(0, 0)

        def pair_body(i, acc):
            k = 2 * i
            issue(k + 1, 1)
            wait(0)
            acc = compute(k, 0, acc)
            issue(k + 2, 0)
            wait(1)
            return compute(k + 1, 1, acc)

        if n_chunks % 2 == 1:
            acc = lax.fori_loop(0, (n_chunks - 1) // 2, pair_body,
                                jnp.zeros((_L,), jnp.float32))
            wait(0)
            acc = compute(n_chunks - 1, 0, acc)
        else:
            acc = lax.fori_loop(0, (n_chunks - 2) // 2, pair_body,
                                jnp.zeros((_L,), jnp.float32))
            issue(n_chunks - 1, 1)
            wait(0)
            acc = compute(n_chunks - 2, 0, acc)
            wait(1)
            acc = compute(n_chunks - 1, 1, acc)
        acc_v[...] = acc
        pltpu.sync_copy(acc_v, out_hbm.at[wid])

    return tri_loss


def kernel(embeddings, emc_embeddings, mom_embeddings, labels, mom_labels,
           triplets):
    T = triplets.shape[0]
    D = embeddings.shape[1]
    n_chunks = max(3, -(-T // (_NW * _C)))
    Tp = _NW * _C * n_chunks
    idx = jnp.pad(triplets, ((0, Tp - T), (0, 0)))
    f = _make_sc_kernel(T, D, n_chunks)
    partial = f(embeddings, emc_embeddings, mom_embeddings,
                idx[:, 0], idx[:, 1], idx[:, 2])
    loss = jnp.sum(partial) / jnp.float32(T)
    return (loss, jnp.asarray(T, dtype=jnp.int32))


# speedup vs baseline: 1.0665x; 1.0665x over previous
import functools

import jax
import jax.numpy as jnp
from jax import lax
from jax.experimental import pallas as pl
from jax.experimental.pallas import tpu as pltpu
from jax.experimental.pallas import tpu_sc as plsc

_MARGIN = 0.2
_NC, _NS, _L = 2, 16, 16
_NW = _NC * _NS
_C = 128


@functools.lru_cache(maxsize=None)
def _make_sc_kernel(T, W, n_chunks):
    assert n_chunks % 2 == 1 and n_chunks >= 3
    n_per_w = n_chunks * _C
    mesh = plsc.VectorSubcoreMesh(core_axis_name="c", subcore_axis_name="s")

    @functools.partial(
        pl.kernel,
        out_type=jax.ShapeDtypeStruct((_NW, _L), jnp.float32),
        mesh=mesh,
        compiler_params=pltpu.CompilerParams(needs_layout_passes=False,
                                             use_tc_tiling_on_sc=False),
        scratch_types=[
            pltpu.VMEM((2, _C), jnp.int32),
            pltpu.VMEM((2, _C), jnp.int32),
            pltpu.VMEM((2, _C), jnp.int32),
            pltpu.VMEM((2, _C, W), jnp.int32),
            pltpu.VMEM((2, _C, W), jnp.int32),
            pltpu.VMEM((2, _C, W), jnp.int32),
            pltpu.VMEM((_L,), jnp.float32),
            pltpu.SemaphoreType.DMA,
            pltpu.SemaphoreType.DMA,
        ],
    )
    def tri_loss(emb_hbm, emc_hbm, mom_hbm, ia_hbm, ip_hbm, in_hbm, out_hbm,
                 ia_v, ip_v, in_v, ra_v, rp_v, rn_v, acc_v, sem0, sem1):
        wid = lax.axis_index("s") * _NC + lax.axis_index("c")
        base_w = wid * n_per_w
        lanes = lax.iota(jnp.int32, _L)
        sems = (sem0, sem1)

        def issue(k, b):
            base = base_w + k * _C
            pltpu.sync_copy(ia_hbm.at[pl.ds(base, _C)], ia_v.at[b])
            pltpu.sync_copy(ip_hbm.at[pl.ds(base, _C)], ip_v.at[b])
            pltpu.sync_copy(in_hbm.at[pl.ds(base, _C)], in_v.at[b])
            pltpu.make_async_copy(emb_hbm.at[ia_v.at[b]], ra_v.at[b],
                                  sems[b]).start()
            pltpu.make_async_copy(emc_hbm.at[ip_v.at[b]], rp_v.at[b],
                                  sems[b]).start()
            pltpu.make_async_copy(mom_hbm.at[in_v.at[b]], rn_v.at[b],
                                  sems[b]).start()

        def wait(b):
            pltpu.make_async_copy(emb_hbm.at[ia_v.at[b]], ra_v.at[b],
                                  sems[b]).wait()
            pltpu.make_async_copy(emc_hbm.at[ip_v.at[b]], rp_v.at[b],
                                  sems[b]).wait()
            pltpu.make_async_copy(mom_hbm.at[in_v.at[b]], rn_v.at[b],
                                  sems[b]).wait()

        himask = jnp.full((_L,), -0x10000, jnp.int32)

        def sqacc_halves(dw):
            lo = plsc.bitcast(dw << 16, jnp.float32)
            hi = plsc.bitcast(dw & himask, jnp.float32)
            return lo * lo, hi * hi

        def compute(k, b, acc):
            base = base_w + k * _C
            ra, rp, rn = ra_v.at[b], rp_v.at[b], rn_v.at[b]

            def group_body(g, acc):
                row = g * _L + lanes
                ap = [jnp.zeros((_L,), jnp.float32) for _ in range(4)]
                an = [jnp.zeros((_L,), jnp.float32) for _ in range(4)]
                for w in range(W):
                    widx = (lanes + w) & (W - 1)
                    va = plsc.load_gather(ra, [row, widx])
                    vp = plsc.load_gather(rp, [row, widx])
                    vn = plsc.load_gather(rn, [row, widx])
                    a16 = plsc.bitcast(va, jnp.bfloat16)
                    p16 = plsc.bitcast(vp, jnp.bfloat16)
                    n16 = plsc.bitcast(vn, jnp.bfloat16)
                    dpw = plsc.bitcast(a16 - p16, jnp.int32)
                    dnw = plsc.bitcast(a16 - n16, jnp.int32)
                    j = w & 1
                    sp_lo, sp_hi = sqacc_halves(dpw)
                    sn_lo, sn_hi = sqacc_halves(dnw)
                    ap[j] = ap[j] + sp_lo
                    ap[2 + j] = ap[2 + j] + sp_hi
                    an[j] = an[j] + sn_lo
                    an[2 + j] = an[2 + j] + sn_hi
                dd = ((ap[0] - an[0]) + (ap[1] - an[1])) + \
                     ((ap[2] - an[2]) + (ap[3] - an[3]))
                dloss = jnp.maximum(dd + _MARGIN, 0.0)
                valid = (base + row) < T
                return acc + jnp.where(valid, dloss, 0.0)

            return lax.fori_loop(0, _C // _L, group_body, acc)

        issue(0, 0)

        def pair_body(i, acc):
            k = 2 * i
            issue(k + 1, 1)
            wait(0)
            acc = compute(k, 0, acc)
            issue(k + 2, 0)
            wait(1)
            return compute(k + 1, 1, acc)

        acc = lax.fori_loop(0, (n_chunks - 1) // 2, pair_body,
                            jnp.zeros((_L,), jnp.float32))
        wait(0)
        acc = compute(n_chunks - 1, 0, acc)
        acc_v[...] = acc
        pltpu.sync_copy(acc_v, out_hbm.at[wid])

    return tri_loss


def _pack_bf16(table):
    b, d = table.shape
    t16 = table.astype(jnp.bfloat16).reshape(b, d // 2, 2)
    return jax.lax.bitcast_convert_type(t16, jnp.int32)


def kernel(embeddings, emc_embeddings, mom_embeddings, labels, mom_labels,
           triplets):
    T = triplets.shape[0]
    D = embeddings.shape[1]
    n_chunks = -(-T // (_NW * _C))
    if n_chunks % 2 == 0:
        n_chunks += 1
    Tp = _NW * _C * n_chunks
    idx = jnp.pad(triplets, ((0, Tp - T), (0, 0)))
    f = _make_sc_kernel(T, D // 2, n_chunks)
    partial = f(_pack_bf16(embeddings), _pack_bf16(emc_embeddings),
                _pack_bf16(mom_embeddings),
                idx[:, 0], idx[:, 1], idx[:, 2])
    loss = jnp.sum(partial) / jnp.float32(T)
    return (loss, jnp.asarray(T, dtype=jnp.int32))


# speedup vs baseline: 1.1475x; 1.0760x over previous
import functools

import jax
import jax.numpy as jnp
from jax import lax
from jax.experimental import pallas as pl
from jax.experimental.pallas import tpu as pltpu
from jax.experimental.pallas import tpu_sc as plsc

_MARGIN = 0.2
_NC, _NS, _L = 2, 16, 16
_NW = _NC * _NS
_C = 128


@functools.lru_cache(maxsize=None)
def _make_sc_kernel(T, W, n_chunks):
    assert n_chunks % 2 == 1 and n_chunks >= 3
    n_per_w = n_chunks * _C
    mesh = plsc.VectorSubcoreMesh(core_axis_name="c", subcore_axis_name="s")

    @functools.partial(
        pl.kernel,
        out_type=jax.ShapeDtypeStruct((_NW, _L), jnp.float32),
        mesh=mesh,
        compiler_params=pltpu.CompilerParams(needs_layout_passes=False,
                                             use_tc_tiling_on_sc=False),
        scratch_types=[
            pltpu.VMEM((2, _C), jnp.int32),
            pltpu.VMEM((2, _C), jnp.int32),
            pltpu.VMEM((2, _C), jnp.int32),
            pltpu.VMEM((2, _C, W), jnp.int32),
            pltpu.VMEM((2, _C, W), jnp.int32),
            pltpu.VMEM((2, _C, W), jnp.int32),
            pltpu.VMEM((_L,), jnp.float32),
            pltpu.SemaphoreType.DMA,
            pltpu.SemaphoreType.DMA,
        ],
    )
    def tri_loss(emb_hbm, emc_hbm, mom_hbm, ia_hbm, ip_hbm, in_hbm, out_hbm,
                 ia_v, ip_v, in_v, ra_v, rp_v, rn_v, acc_v, sem0, sem1):
        wid = lax.axis_index("s") * _NC + lax.axis_index("c")
        base_w = wid * n_per_w
        lanes = lax.iota(jnp.int32, _L)
        sems = (sem0, sem1)

        def issue(k, b):
            base = base_w + k * _C
            pltpu.sync_copy(ia_hbm.at[pl.ds(base, _C)], ia_v.at[b])
            pltpu.sync_copy(ip_hbm.at[pl.ds(base, _C)], ip_v.at[b])
            pltpu.sync_copy(in_hbm.at[pl.ds(base, _C)], in_v.at[b])
            pltpu.make_async_copy(emb_hbm.at[ia_v.at[b]], ra_v.at[b],
                                  sems[b]).start()
            pltpu.make_async_copy(emc_hbm.at[ip_v.at[b]], rp_v.at[b],
                                  sems[b]).start()
            pltpu.make_async_copy(mom_hbm.at[in_v.at[b]], rn_v.at[b],
                                  sems[b]).start()

        def wait(b):
            pltpu.make_async_copy(emb_hbm.at[ia_v.at[b]], ra_v.at[b],
                                  sems[b]).wait()
            pltpu.make_async_copy(emc_hbm.at[ip_v.at[b]], rp_v.at[b],
                                  sems[b]).wait()
            pltpu.make_async_copy(mom_hbm.at[in_v.at[b]], rn_v.at[b],
                                  sems[b]).wait()

        himask = jnp.full((_L,), -0x10000, jnp.int32)

        def sqacc_halves(dw):
            lo = plsc.bitcast(dw << 16, jnp.float32)
            hi = plsc.bitcast(dw & himask, jnp.float32)
            return lo * lo, hi * hi

        def compute(k, b, acc):
            base = base_w + k * _C
            ra, rp, rn = ra_v.at[b], rp_v.at[b], rn_v.at[b]

            def group_body(g, acc):
                row = g * _L + lanes
                ap = [jnp.zeros((_L,), jnp.float32) for _ in range(4)]
                an = [jnp.zeros((_L,), jnp.float32) for _ in range(4)]
                for w in range(W):
                    widx = (lanes + w) & (W - 1)
                    va = plsc.load_gather(ra, [row, widx])
                    vp = plsc.load_gather(rp, [row, widx])
                    vn = plsc.load_gather(rn, [row, widx])
                    a16 = plsc.bitcast(va, jnp.bfloat16)
                    p16 = plsc.bitcast(vp, jnp.bfloat16)
                    n16 = plsc.bitcast(vn, jnp.bfloat16)
                    dpw = plsc.bitcast(a16 - p16, jnp.int32)
                    dnw = plsc.bitcast(a16 - n16, jnp.int32)
                    j = w & 1
                    sp_lo, sp_hi = sqacc_halves(dpw)
                    sn_lo, sn_hi = sqacc_halves(dnw)
                    ap[j] = ap[j] + sp_lo
                    ap[2 + j] = ap[2 + j] + sp_hi
                    an[j] = an[j] + sn_lo
                    an[2 + j] = an[2 + j] + sn_hi
                dd = ((ap[0] - an[0]) + (ap[1] - an[1])) + \
                     ((ap[2] - an[2]) + (ap[3] - an[3]))
                dloss = jnp.maximum(dd + _MARGIN, 0.0)
                valid = (base + row) < T
                return acc + jnp.where(valid, dloss, 0.0)

            return lax.fori_loop(0, _C // _L, group_body, acc)

        issue(0, 0)

        def pair_body(i, acc):
            k = 2 * i
            issue(k + 1, 1)
            wait(0)
            acc = compute(k, 0, acc)
            issue(k + 2, 0)
            wait(1)
            return compute(k + 1, 1, acc)

        acc = lax.fori_loop(0, (n_chunks - 1) // 2, pair_body,
                            jnp.zeros((_L,), jnp.float32))
        wait(0)
        acc = compute(n_chunks - 1, 0, acc)
        acc_v[...] = acc
        pltpu.sync_copy(acc_v, out_hbm.at[wid])

    return tri_loss


def _pack_bf16_pairs(table):
    d = table.shape[1]
    bits = jax.lax.bitcast_convert_type(table, jnp.uint32)

    def rnd(x):
        return x + 0x7FFF + ((x >> 16) & 1)

    lo = rnd(bits[:, : d // 2]) >> 16
    hi = rnd(bits[:, d // 2:]) & jnp.uint32(0xFFFF0000)
    return jax.lax.bitcast_convert_type(hi | lo, jnp.int32)


def kernel(embeddings, emc_embeddings, mom_embeddings, labels, mom_labels,
           triplets):
    T = triplets.shape[0]
    D = embeddings.shape[1]
    n_chunks = -(-T // (_NW * _C))
    if n_chunks % 2 == 0:
        n_chunks += 1
    Tp = _NW * _C * n_chunks
    idx = jnp.pad(triplets, ((0, Tp - T), (0, 0)))
    f = _make_sc_kernel(T, D // 2, n_chunks)
    partial = f(_pack_bf16_pairs(embeddings),
                _pack_bf16_pairs(emc_embeddings),
                _pack_bf16_pairs(mom_embeddings),
                idx[:, 0], idx[:, 1], idx[:, 2])
    loss = jnp.sum(partial) / jnp.float32(T)
    return (loss, jnp.asarray(T, dtype=jnp.int32))
